# Initial kernel scaffold; baseline (speedup 1.0000x reference)
import jax
import jax.numpy as jnp
from jax.experimental import pallas as pl


def kernel(embs, prnt_indices, lba_out, rnn_out, W):
    raise NotImplementedError("write your pallas kernel here")


# speedup vs baseline: 23.1839x; 23.1839x over previous
import jax
import jax.numpy as jnp
from jax.experimental import pallas as pl

B, L, E, R = 1024, 200, 128, 128
EPS = 1e-7
BB = 16


def _cba_kernel(p_ref, lba_ref, embs_ref, rnn_ref, w_ref, out_ref):
    wsum = jnp.sum(w_ref[...], axis=1)
    w1 = wsum[:R]
    w2 = wsum[R:]
    s1 = jnp.sum(lba_ref[...] * w1[None, None, :], axis=-1)
    s2 = jnp.sum(embs_ref[...] * w2[None, None, :], axis=-1)
    p = p_ref[...]
    iota = jax.lax.broadcasted_iota(jnp.int32, (1, 1, L), 2)
    onehot = p[:, :, None] == iota
    g = jnp.sum(jnp.where(onehot, s1[:, None, :], 0.0), axis=-1)
    a = jnp.exp(jnp.tanh(g + s2))
    a = a / (jnp.sum(a, axis=1, keepdims=True) + EPS)
    out_ref[...] = jnp.sum(rnn_ref[...] * a[:, :, None], axis=1)


def kernel(embs, prnt_indices, lba_out, rnn_out, W):
    grid = (B // BB,)
    return pl.pallas_call(
        _cba_kernel,
        grid=grid,
        in_specs=[
            pl.BlockSpec((BB, L), lambda i: (i, 0)),
            pl.BlockSpec((BB, L, R), lambda i: (i, 0, 0)),
            pl.BlockSpec((BB, L, E), lambda i: (i, 0, 0)),
            pl.BlockSpec((BB, L, R), lambda i: (i, 0, 0)),
            pl.BlockSpec((E + R, R), lambda i: (0, 0)),
        ],
        out_specs=pl.BlockSpec((BB, R), lambda i: (i, 0)),
        out_shape=jax.ShapeDtypeStruct((B, R), jnp.float32),
    )(prnt_indices, lba_out, embs, rnn_out, W)


# speedup vs baseline: 35.5410x; 1.5330x over previous
import jax
import jax.numpy as jnp
from jax.experimental import pallas as pl

B, L, E, R = 1024, 200, 128, 128
EPS = 1e-7
BB = 32


def _cba_kernel(p_ref, lba_ref, embs_ref, rnn_ref, w_ref, out_ref):
    wsum = jnp.sum(w_ref[...], axis=1)
    w1b = jnp.broadcast_to(wsum[:R].reshape(1, 1, R), (BB, 1, R))
    w2b = jnp.broadcast_to(wsum[R:].reshape(1, 1, E), (BB, 1, E))
    s1 = jax.lax.dot_general(
        w1b, lba_ref[...], (((2,), (2,)), ((0,), (0,))),
        preferred_element_type=jnp.float32)
    s2 = jax.lax.dot_general(
        w2b, embs_ref[...], (((2,), (2,)), ((0,), (0,))),
        preferred_element_type=jnp.float32)
    p = p_ref[...]
    s1d = s1[:, 0, :]
    s1a = s1d[:, :128]
    s1b = jnp.pad(s1d[:, 128:], ((0, 0), (0, 256 - L)))
    pa = jnp.minimum(p, 127)
    pb = jnp.minimum(jnp.maximum(p - 128, 0), 127)
    ga = jnp.take_along_axis(s1a, pa, axis=1)
    gb = jnp.take_along_axis(s1b, pb, axis=1)
    g = jnp.where(p < 128, ga, gb)
    a = jnp.exp(jnp.tanh(g + s2[:, 0, :]))
    num = jax.lax.dot_general(
        a[:, None, :], rnn_ref[...], (((2,), (1,)), ((0,), (0,))),
        preferred_element_type=jnp.float32)
    denom = jnp.sum(a, axis=1)[:, None] + EPS
    out_ref[...] = num[:, 0, :] / denom


def kernel(embs, prnt_indices, lba_out, rnn_out, W):
    grid = (B // BB,)
    return pl.pallas_call(
        _cba_kernel,
        grid=grid,
        in_specs=[
            pl.BlockSpec((BB, L), lambda i: (i, 0)),
            pl.BlockSpec((BB, L, R), lambda i: (i, 0, 0)),
            pl.BlockSpec((BB, L, E), lambda i: (i, 0, 0)),
            pl.BlockSpec((BB, L, R), lambda i: (i, 0, 0)),
            pl.BlockSpec((E + R, R), lambda i: (0, 0)),
        ],
        out_specs=pl.BlockSpec((BB, R), lambda i: (i, 0)),
        out_shape=jax.ShapeDtypeStruct((B, R), jnp.float32),
    )(prnt_indices, lba_out, embs, rnn_out, W)


# speedup vs baseline: 38.1376x; 1.0731x over previous
import jax
import jax.numpy as jnp
from jax.experimental import pallas as pl

B, L, E, R = 1024, 200, 128, 128
EPS = 1e-7
BB = 64


def _cba_kernel(p_ref, lba_ref, embs_ref, rnn_ref, w_ref, out_ref):
    wsum = jnp.sum(w_ref[...], axis=1)
    w1b = jnp.broadcast_to(wsum[:R].reshape(1, 1, R), (BB, 1, R))
    w2b = jnp.broadcast_to(wsum[R:].reshape(1, 1, E), (BB, 1, E))
    s1 = jax.lax.dot_general(
        w1b, lba_ref[...], (((2,), (2,)), ((0,), (0,))),
        preferred_element_type=jnp.float32)
    s2 = jax.lax.dot_general(
        w2b, embs_ref[...], (((2,), (2,)), ((0,), (0,))),
        preferred_element_type=jnp.float32)
    p = p_ref[...]
    s1d = s1[:, 0, :]
    s1a = s1d[:, :128]
    s1b = jnp.pad(s1d[:, 128:], ((0, 0), (0, 256 - L)))
    pa = jnp.minimum(p, 127)
    pb = jnp.minimum(jnp.maximum(p - 128, 0), 127)
    ga = jnp.take_along_axis(s1a, pa, axis=1)
    gb = jnp.take_along_axis(s1b, pb, axis=1)
    g = jnp.where(p < 128, ga, gb)
    a = jnp.exp(jnp.tanh(g + s2[:, 0, :]))
    num = jax.lax.dot_general(
        a[:, None, :], rnn_ref[...], (((2,), (1,)), ((0,), (0,))),
        preferred_element_type=jnp.float32)
    denom = jnp.sum(a, axis=1)[:, None] + EPS
    out_ref[...] = num[:, 0, :] / denom


def kernel(embs, prnt_indices, lba_out, rnn_out, W):
    grid = (B // BB,)
    return pl.pallas_call(
        _cba_kernel,
        grid=grid,
        in_specs=[
            pl.BlockSpec((BB, L), lambda i: (i, 0)),
            pl.BlockSpec((BB, L, R), lambda i: (i, 0, 0)),
            pl.BlockSpec((BB, L, E), lambda i: (i, 0, 0)),
            pl.BlockSpec((BB, L, R), lambda i: (i, 0, 0)),
            pl.BlockSpec((E + R, R), lambda i: (0, 0)),
        ],
        out_specs=pl.BlockSpec((BB, R), lambda i: (i, 0)),
        out_shape=jax.ShapeDtypeStruct((B, R), jnp.float32),
    )(prnt_indices, lba_out, embs, rnn_out, W)
